# baseline (speedup 1.0000x reference)
import functools

import jax
import jax.numpy as jnp
from jax import lax
from jax.experimental import pallas as pl
from jax.experimental.pallas import tpu as pltpu
from jax.experimental.pallas import tpu_sc as plsc

N = 2048
C = 768
E = 8
H = 2048
HC = H // 2
CAPACITY = 640
NSEG = 2 * E
XRB = 640
ROW0 = XRB
XE_ROWS = ROW0 + NSEG * CAPACITY
NW = 32
TPW = N // NW


def _router_body(x_ref, gw_ref, d0_ref, d1_ref, wx0_ref, wx1_ref,
                 c1_ref, c2_ref, aux_ref, z_ref):
    xf = x_ref[...]
    gw = gw_ref[...]
    logits = jax.lax.dot_general(
        xf, gw, (((1,), (1,)), ((), ())), preferred_element_type=jnp.float32)
    m = jnp.max(logits, axis=1, keepdims=True)
    ex = jnp.exp(logits - m)
    gates = ex / jnp.sum(ex, axis=1, keepdims=True)

    lane = jax.lax.broadcasted_iota(jnp.int32, (N, E), 1)
    top1_v = jnp.max(gates, axis=1, keepdims=True)
    top1_i = jnp.min(jnp.where(gates == top1_v, lane, E), axis=1,
                     keepdims=True)
    masked = jnp.where(lane == top1_i, -jnp.inf, gates)
    top2_v = jnp.max(masked, axis=1, keepdims=True)
    top2_i = jnp.min(jnp.where(masked == top2_v, lane, E), axis=1,
                     keepdims=True)

    me = jnp.sum(gates, axis=0, keepdims=True) * (1.0 / N)
    onehot1 = (lane == top1_i).astype(jnp.float32)
    ce = jnp.sum(onehot1, axis=0, keepdims=True) * (1.0 / N)
    aux_ref[...] = E * jnp.sum(me * ce, axis=1, keepdims=True)
    zrow = jnp.sum(logits * logits, axis=0, keepdims=True)
    z_ref[...] = jnp.sum(zrow, axis=1, keepdims=True) * (1.0 / (N * E))

    r = jax.lax.broadcasted_iota(jnp.int32, (N, N), 0)
    ccol = jax.lax.broadcasted_iota(jnp.int32, (N, N), 1)
    tri = (r >= ccol).astype(jnp.bfloat16)
    onehot2 = (lane == top2_i).astype(jnp.float32)
    ranks1 = jax.lax.dot_general(
        tri, onehot1.astype(jnp.bfloat16), (((1,), (0,)), ((), ())),
        preferred_element_type=jnp.float32)
    ranks2 = jax.lax.dot_general(
        tri, onehot2.astype(jnp.bfloat16), (((1,), (0,)), ((), ())),
        preferred_element_type=jnp.float32)

    rt1 = jnp.sum(ranks1 * onehot1, axis=1, keepdims=True).astype(jnp.int32)
    rt2 = jnp.sum(ranks2 * onehot2, axis=1, keepdims=True).astype(jnp.int32)
    kept1 = rt1 <= CAPACITY
    kept2 = rt2 <= CAPACITY
    row1 = ROW0 + (2 * top1_i) * CAPACITY + jnp.minimum(rt1 - 1, CAPACITY - 1)
    row2 = ROW0 + (2 * top2_i + 1) * CAPACITY + jnp.minimum(rt2 - 1,
                                                            CAPACITY - 1)
    d0_ref[...] = jnp.where(kept1, row1, 0)
    d1_ref[...] = jnp.where(kept2, row2, 0)
    wx0_ref[...] = jnp.broadcast_to(jnp.where(kept1, top1_v, 0.0), (N, 16))
    wx1_ref[...] = jnp.broadcast_to(jnp.where(kept2, top2_v, 0.0), (N, 16))

    c1_ref[...] = jnp.minimum(jnp.sum(onehot1, axis=0, keepdims=True),
                              float(CAPACITY)).astype(jnp.int32)
    c2_ref[...] = jnp.minimum(jnp.sum(onehot2, axis=0, keepdims=True),
                              float(CAPACITY)).astype(jnp.int32)


def _router(xf, gate_w):
    return pl.pallas_call(
        _router_body,
        out_shape=(
            jax.ShapeDtypeStruct((N, 1), jnp.int32),
            jax.ShapeDtypeStruct((N, 1), jnp.int32),
            jax.ShapeDtypeStruct((N, 16), jnp.float32),
            jax.ShapeDtypeStruct((N, 16), jnp.float32),
            jax.ShapeDtypeStruct((1, E), jnp.int32),
            jax.ShapeDtypeStruct((1, E), jnp.int32),
            jax.ShapeDtypeStruct((1, 1), jnp.float32),
            jax.ShapeDtypeStruct((1, 1), jnp.float32),
        ),
    )(xf, gate_w)


def _dispatch(xf, dest0, dest1):
    mesh = plsc.VectorSubcoreMesh(core_axis_name="c", subcore_axis_name="s")

    @functools.partial(
        pl.kernel, mesh=mesh,
        out_type=jax.ShapeDtypeStruct((XE_ROWS, C), jnp.float32),
        scratch_types=[
            pltpu.VMEM((TPW,), jnp.int32),
            pltpu.VMEM((TPW,), jnp.int32),
            pltpu.VMEM((TPW, C), jnp.float32),
            pltpu.SemaphoreType.DMA,
            pltpu.SemaphoreType.DMA,
        ],
    )
    def k(xf_hbm, d0_hbm, d1_hbm, xe_hbm,
          d0_v, d1_v, xbuf, sem0, sem1):
        wid = lax.axis_index("s") * 2 + lax.axis_index("c")
        base = wid * TPW
        pltpu.sync_copy(d0_hbm.at[wid], d0_v)
        pltpu.sync_copy(d1_hbm.at[wid], d1_v)
        pltpu.sync_copy(xf_hbm.at[pl.ds(base, TPW)], xbuf)
        cp0 = pltpu.async_copy(xbuf, xe_hbm.at[d0_v], sem0)
        cp1 = pltpu.async_copy(xbuf, xe_hbm.at[d1_v], sem1)
        cp0.wait()
        cp1.wait()

    return k(xf, dest0, dest1)


_NB = XE_ROWS // XRB


def _expert_body(c1_ref, c2_ref, xe_ref, wg_ref, wu_ref, wd_ref, y_ref):
    b = pl.program_id(0)
    h = pl.program_id(1)

    @pl.when((b == 0) & (h == 0))
    def _():
        y_ref[...] = jnp.zeros((XRB, C), jnp.float32)

    @pl.when(b > 0)
    def _():
        seg = b - 1
        e = seg // 2
        kslot = seg % 2
        cnt = jnp.where(kslot == 0, c1_ref[e], c2_ref[e])

        @pl.when(cnt > 0)
        def _():
            xb = xe_ref[...].astype(jnp.bfloat16)
            hg = jax.lax.dot_general(
                xb, wg_ref[0].astype(jnp.bfloat16), (((1,), (1,)), ((), ())),
                preferred_element_type=jnp.float32)
            hu = jax.lax.dot_general(
                xb, wu_ref[0].astype(jnp.bfloat16), (((1,), (1,)), ((), ())),
                preferred_element_type=jnp.float32)
            silu = hg / (1.0 + jnp.exp(-hg))
            hidden = (silu * hu).astype(jnp.bfloat16)
            yc = jax.lax.dot_general(
                hidden, wd_ref[0].astype(jnp.bfloat16),
                (((1,), (1,)), ((), ())),
                preferred_element_type=jnp.float32)

            @pl.when(h == 0)
            def _():
                y_ref[...] = yc

            @pl.when(h == 1)
            def _():
                y_ref[...] = y_ref[...] + yc


def _experts(c1, c2, xe, wg, wu, wd):
    grid_spec = pltpu.PrefetchScalarGridSpec(
        num_scalar_prefetch=2,
        grid=(_NB, 2),
        in_specs=[
            pl.BlockSpec((XRB, C), lambda b, h, c1, c2: (b, 0)),
            pl.BlockSpec((1, HC, C),
                         lambda b, h, c1, c2: (jnp.maximum(b - 1, 0) // 2,
                                               h, 0)),
            pl.BlockSpec((1, HC, C),
                         lambda b, h, c1, c2: (jnp.maximum(b - 1, 0) // 2,
                                               h, 0)),
            pl.BlockSpec((1, C, HC),
                         lambda b, h, c1, c2: (jnp.maximum(b - 1, 0) // 2,
                                               0, h)),
        ],
        out_specs=pl.BlockSpec((XRB, C), lambda b, h, c1, c2: (b, 0)),
    )
    return pl.pallas_call(
        _expert_body,
        grid_spec=grid_spec,
        out_shape=jax.ShapeDtypeStruct((XE_ROWS, C), jnp.float32),
    )(c1, c2, xe, wg, wu, wd)


def _combine(y, row0, row1, wx0, wx1):
    mesh = plsc.VectorSubcoreMesh(core_axis_name="c", subcore_axis_name="s")

    @functools.partial(
        pl.kernel, mesh=mesh,
        out_type=jax.ShapeDtypeStruct((N, C), jnp.float32),
        scratch_types=[
            pltpu.VMEM((TPW,), jnp.int32),
            pltpu.VMEM((TPW,), jnp.int32),
            pltpu.VMEM((TPW, 16), jnp.float32),
            pltpu.VMEM((TPW, 16), jnp.float32),
            pltpu.VMEM((TPW, C), jnp.float32),
            pltpu.VMEM((TPW, C), jnp.float32),
            pltpu.SemaphoreType.DMA,
            pltpu.SemaphoreType.DMA,
        ],
    )
    def k(y_hbm, r0_hbm, r1_hbm, wx0_hbm, wx1_hbm, out_hbm,
          r0_v, r1_v, wx0_v, wx1_v, buf0, buf1, sem0, sem1):
        wid = lax.axis_index("s") * 2 + lax.axis_index("c")
        base = wid * TPW
        pltpu.sync_copy(r0_hbm.at[wid], r0_v)
        pltpu.sync_copy(r1_hbm.at[wid], r1_v)
        pltpu.sync_copy(wx0_hbm.at[wid], wx0_v)
        pltpu.sync_copy(wx1_hbm.at[wid], wx1_v)
        cp0 = pltpu.async_copy(y_hbm.at[r0_v], buf0, sem0)
        cp1 = pltpu.async_copy(y_hbm.at[r1_v], buf1, sem1)
        cp0.wait()
        cp1.wait()

        def row_body(i, carry):
            wv0 = wx0_v[i, :]
            wv1 = wx1_v[i, :]
            for j in range(C // 16):
                sl = pl.ds(j * 16, 16)
                buf0[i, sl] = wv0 * buf0[i, sl] + wv1 * buf1[i, sl]
            return carry

        lax.fori_loop(0, TPW, row_body, 0)
        pltpu.sync_copy(buf0, out_hbm.at[pl.ds(base, TPW)])

    return k(y, row0, row1, wx0, wx1)


def kernel(x, gate_w, wg, wu, wd):
    b, t, c = x.shape
    xf = x.reshape(b * t, c)
    d0, d1, wx0, wx1, c1, c2, aux, z = _router(xf, gate_w)

    xe = _dispatch(xf, d0.reshape(NW, TPW), d1.reshape(NW, TPW))

    y = _experts(c1.reshape(E), c2.reshape(E), xe, wg, wu, wd)

    out = _combine(y, d0.reshape(NW, TPW), d1.reshape(NW, TPW),
                   wx0.reshape(NW, TPW, 16), wx1.reshape(NW, TPW, 16))
    return out.reshape(b, t, c), aux[0, 0], z[0, 0]


# speedup vs baseline: 1.1216x; 1.1216x over previous
import functools

import jax
import jax.numpy as jnp
from jax import lax
from jax.experimental import pallas as pl
from jax.experimental.pallas import tpu as pltpu
from jax.experimental.pallas import tpu_sc as plsc

N = 2048
C = 768
E = 8
H = 2048
CAPACITY = 640
NSEG = 2 * E
XRB = 640
ROW0 = XRB
XE_ROWS = ROW0 + NSEG * CAPACITY
NW = 32
TPW = N // NW


def _router_body(x_ref, gw_ref, d0_ref, d1_ref, wx0_ref, wx1_ref,
                 c1_ref, c2_ref, aux_ref, z_ref):
    xf = x_ref[...]
    gw = gw_ref[...]
    logits = jax.lax.dot_general(
        xf, gw, (((1,), (1,)), ((), ())), preferred_element_type=jnp.float32)
    m = jnp.max(logits, axis=1, keepdims=True)
    ex = jnp.exp(logits - m)
    gates = ex / jnp.sum(ex, axis=1, keepdims=True)

    lane = jax.lax.broadcasted_iota(jnp.int32, (N, E), 1)
    top1_v = jnp.max(gates, axis=1, keepdims=True)
    top1_i = jnp.min(jnp.where(gates == top1_v, lane, E), axis=1,
                     keepdims=True)
    masked = jnp.where(lane == top1_i, -jnp.inf, gates)
    top2_v = jnp.max(masked, axis=1, keepdims=True)
    top2_i = jnp.min(jnp.where(masked == top2_v, lane, E), axis=1,
                     keepdims=True)

    me = jnp.sum(gates, axis=0, keepdims=True) * (1.0 / N)
    onehot1 = (lane == top1_i).astype(jnp.float32)
    ce = jnp.sum(onehot1, axis=0, keepdims=True) * (1.0 / N)
    aux_ref[...] = E * jnp.sum(me * ce, axis=1, keepdims=True)
    zrow = jnp.sum(logits * logits, axis=0, keepdims=True)
    z_ref[...] = jnp.sum(zrow, axis=1, keepdims=True) * (1.0 / (N * E))

    r = jax.lax.broadcasted_iota(jnp.int32, (N, N), 0)
    ccol = jax.lax.broadcasted_iota(jnp.int32, (N, N), 1)
    tri = (r >= ccol).astype(jnp.bfloat16)
    onehot2 = (lane == top2_i).astype(jnp.float32)
    ranks1 = jax.lax.dot_general(
        tri, onehot1.astype(jnp.bfloat16), (((1,), (0,)), ((), ())),
        preferred_element_type=jnp.float32)
    ranks2 = jax.lax.dot_general(
        tri, onehot2.astype(jnp.bfloat16), (((1,), (0,)), ((), ())),
        preferred_element_type=jnp.float32)

    rt1 = jnp.sum(ranks1 * onehot1, axis=1, keepdims=True).astype(jnp.int32)
    rt2 = jnp.sum(ranks2 * onehot2, axis=1, keepdims=True).astype(jnp.int32)
    kept1 = rt1 <= CAPACITY
    kept2 = rt2 <= CAPACITY
    row1 = ROW0 + (2 * top1_i) * CAPACITY + jnp.minimum(rt1 - 1, CAPACITY - 1)
    row2 = ROW0 + (2 * top2_i + 1) * CAPACITY + jnp.minimum(rt2 - 1,
                                                            CAPACITY - 1)
    d0_ref[...] = jnp.where(kept1, row1, 0)
    d1_ref[...] = jnp.where(kept2, row2, 0)
    wx0_ref[...] = jnp.broadcast_to(jnp.where(kept1, top1_v, 0.0), (N, 16))
    wx1_ref[...] = jnp.broadcast_to(jnp.where(kept2, top2_v, 0.0), (N, 16))

    c1_ref[...] = jnp.minimum(jnp.sum(onehot1, axis=0, keepdims=True),
                              float(CAPACITY)).astype(jnp.int32)
    c2_ref[...] = jnp.minimum(jnp.sum(onehot2, axis=0, keepdims=True),
                              float(CAPACITY)).astype(jnp.int32)


def _router(xf, gate_w):
    return pl.pallas_call(
        _router_body,
        out_shape=(
            jax.ShapeDtypeStruct((N, 1), jnp.int32),
            jax.ShapeDtypeStruct((N, 1), jnp.int32),
            jax.ShapeDtypeStruct((N, 16), jnp.float32),
            jax.ShapeDtypeStruct((N, 16), jnp.float32),
            jax.ShapeDtypeStruct((1, E), jnp.int32),
            jax.ShapeDtypeStruct((1, E), jnp.int32),
            jax.ShapeDtypeStruct((1, 1), jnp.float32),
            jax.ShapeDtypeStruct((1, 1), jnp.float32),
        ),
    )(xf, gate_w)


def _dispatch(xf, dest0, dest1):
    mesh = plsc.VectorSubcoreMesh(core_axis_name="c", subcore_axis_name="s")

    @functools.partial(
        pl.kernel, mesh=mesh,
        out_type=jax.ShapeDtypeStruct((XE_ROWS, C), jnp.float32),
        scratch_types=[
            pltpu.VMEM((TPW,), jnp.int32),
            pltpu.VMEM((TPW,), jnp.int32),
            pltpu.VMEM((TPW, C), jnp.float32),
            pltpu.SemaphoreType.DMA,
            pltpu.SemaphoreType.DMA,
        ],
    )
    def k(xf_hbm, d0_hbm, d1_hbm, xe_hbm,
          d0_v, d1_v, xbuf, sem0, sem1):
        wid = lax.axis_index("s") * 2 + lax.axis_index("c")
        base = wid * TPW
        pltpu.sync_copy(d0_hbm.at[wid], d0_v)
        pltpu.sync_copy(d1_hbm.at[wid], d1_v)
        pltpu.sync_copy(xf_hbm.at[pl.ds(base, TPW)], xbuf)
        cp0 = pltpu.async_copy(xbuf, xe_hbm.at[d0_v], sem0)
        cp1 = pltpu.async_copy(xbuf, xe_hbm.at[d1_v], sem1)
        cp0.wait()
        cp1.wait()

    return k(xf, dest0, dest1)


_NB = XE_ROWS // XRB


def _expert_body(c1_ref, c2_ref, xe_ref, wg_ref, wu_ref, wd_ref, y_ref):
    b = pl.program_id(0)

    @pl.when(b == 0)
    def _():
        y_ref[...] = jnp.zeros((XRB, C), jnp.float32)

    @pl.when(b > 0)
    def _():
        seg = b - 1
        e = seg // 2
        kslot = seg % 2
        cnt = jnp.where(kslot == 0, c1_ref[e], c2_ref[e])

        @pl.when(cnt > 0)
        def _():
            xb = xe_ref[...].astype(jnp.bfloat16)
            hg = jax.lax.dot_general(
                xb, wg_ref[0].astype(jnp.bfloat16), (((1,), (1,)), ((), ())),
                preferred_element_type=jnp.float32)
            hu = jax.lax.dot_general(
                xb, wu_ref[0].astype(jnp.bfloat16), (((1,), (1,)), ((), ())),
                preferred_element_type=jnp.float32)
            silu = hg / (1.0 + jnp.exp(-hg))
            hidden = (silu * hu).astype(jnp.bfloat16)
            y_ref[...] = jax.lax.dot_general(
                hidden, wd_ref[0].astype(jnp.bfloat16),
                (((1,), (1,)), ((), ())),
                preferred_element_type=jnp.float32)


def _experts(c1, c2, xe, wg, wu, wd):
    grid_spec = pltpu.PrefetchScalarGridSpec(
        num_scalar_prefetch=2,
        grid=(_NB,),
        in_specs=[
            pl.BlockSpec((XRB, C), lambda b, c1, c2: (b, 0)),
            pl.BlockSpec((1, H, C),
                         lambda b, c1, c2: (jnp.maximum(b - 1, 0) // 2, 0, 0)),
            pl.BlockSpec((1, H, C),
                         lambda b, c1, c2: (jnp.maximum(b - 1, 0) // 2, 0, 0)),
            pl.BlockSpec((1, C, H),
                         lambda b, c1, c2: (jnp.maximum(b - 1, 0) // 2, 0, 0)),
        ],
        out_specs=pl.BlockSpec((XRB, C), lambda b, c1, c2: (b, 0)),
    )
    return pl.pallas_call(
        _expert_body,
        grid_spec=grid_spec,
        out_shape=jax.ShapeDtypeStruct((XE_ROWS, C), jnp.float32),
    )(c1, c2, xe, wg, wu, wd)


def _combine(y, row0, row1, wx0, wx1):
    mesh = plsc.VectorSubcoreMesh(core_axis_name="c", subcore_axis_name="s")

    @functools.partial(
        pl.kernel, mesh=mesh,
        out_type=jax.ShapeDtypeStruct((N, C), jnp.float32),
        scratch_types=[
            pltpu.VMEM((TPW,), jnp.int32),
            pltpu.VMEM((TPW,), jnp.int32),
            pltpu.VMEM((TPW, 16), jnp.float32),
            pltpu.VMEM((TPW, 16), jnp.float32),
            pltpu.VMEM((TPW, C), jnp.float32),
            pltpu.VMEM((TPW, C), jnp.float32),
            pltpu.SemaphoreType.DMA,
            pltpu.SemaphoreType.DMA,
        ],
    )
    def k(y_hbm, r0_hbm, r1_hbm, wx0_hbm, wx1_hbm, out_hbm,
          r0_v, r1_v, wx0_v, wx1_v, buf0, buf1, sem0, sem1):
        wid = lax.axis_index("s") * 2 + lax.axis_index("c")
        base = wid * TPW
        pltpu.sync_copy(r0_hbm.at[wid], r0_v)
        pltpu.sync_copy(r1_hbm.at[wid], r1_v)
        pltpu.sync_copy(wx0_hbm.at[wid], wx0_v)
        pltpu.sync_copy(wx1_hbm.at[wid], wx1_v)
        cp0 = pltpu.async_copy(y_hbm.at[r0_v], buf0, sem0)
        cp1 = pltpu.async_copy(y_hbm.at[r1_v], buf1, sem1)
        cp0.wait()
        cp1.wait()

        def row_body(i, carry):
            wv0 = wx0_v[i, :]
            wv1 = wx1_v[i, :]
            for j in range(C // 16):
                sl = pl.ds(j * 16, 16)
                buf0[i, sl] = wv0 * buf0[i, sl] + wv1 * buf1[i, sl]
            return carry

        lax.fori_loop(0, TPW, row_body, 0)
        pltpu.sync_copy(buf0, out_hbm.at[pl.ds(base, TPW)])

    return k(y, row0, row1, wx0, wx1)


def kernel(x, gate_w, wg, wu, wd):
    b, t, c = x.shape
    xf = x.reshape(b * t, c)
    d0, d1, wx0, wx1, c1, c2, aux, z = _router(xf, gate_w)

    xe = _dispatch(xf, d0.reshape(NW, TPW), d1.reshape(NW, TPW))

    y = _experts(c1.reshape(E), c2.reshape(E), xe, wg, wu, wd)

    out = _combine(y, d0.reshape(NW, TPW), d1.reshape(NW, TPW),
                   wx0.reshape(NW, TPW, 16), wx1.reshape(NW, TPW, 16))
    return out.reshape(b, t, c), aux[0, 0], z[0, 0]


# speedup vs baseline: 1.1384x; 1.0150x over previous
import functools

import jax
import jax.numpy as jnp
from jax import lax
from jax.experimental import pallas as pl
from jax.experimental.pallas import tpu as pltpu
from jax.experimental.pallas import tpu_sc as plsc

N = 2048
C = 768
E = 8
H = 2048
CAPACITY = 640
NSEG = 2 * E
XRB = 640
ROW0 = XRB
XE_ROWS = ROW0 + NSEG * CAPACITY
NW = 32
TPW = N // NW


def _router_body(x_ref, gw_ref, d0_ref, d1_ref, wx0_ref, wx1_ref,
                 c1_ref, c2_ref, aux_ref, z_ref):
    xf = x_ref[...]
    gw = gw_ref[...]
    logits = jax.lax.dot_general(
        xf, gw, (((1,), (1,)), ((), ())), preferred_element_type=jnp.float32)
    m = jnp.max(logits, axis=1, keepdims=True)
    ex = jnp.exp(logits - m)
    gates = ex / jnp.sum(ex, axis=1, keepdims=True)

    lane = jax.lax.broadcasted_iota(jnp.int32, (N, E), 1)
    top1_v = jnp.max(gates, axis=1, keepdims=True)
    top1_i = jnp.min(jnp.where(gates == top1_v, lane, E), axis=1,
                     keepdims=True)
    masked = jnp.where(lane == top1_i, -jnp.inf, gates)
    top2_v = jnp.max(masked, axis=1, keepdims=True)
    top2_i = jnp.min(jnp.where(masked == top2_v, lane, E), axis=1,
                     keepdims=True)

    me = jnp.sum(gates, axis=0, keepdims=True) * (1.0 / N)
    onehot1 = (lane == top1_i).astype(jnp.float32)
    ce = jnp.sum(onehot1, axis=0, keepdims=True) * (1.0 / N)
    aux_ref[...] = E * jnp.sum(me * ce, axis=1, keepdims=True)
    zrow = jnp.sum(logits * logits, axis=0, keepdims=True)
    z_ref[...] = jnp.sum(zrow, axis=1, keepdims=True) * (1.0 / (N * E))

    r = jax.lax.broadcasted_iota(jnp.int32, (N, N), 0)
    ccol = jax.lax.broadcasted_iota(jnp.int32, (N, N), 1)
    tri = (r >= ccol).astype(jnp.bfloat16)
    onehot2 = (lane == top2_i).astype(jnp.float32)
    ranks1 = jax.lax.dot_general(
        tri, onehot1.astype(jnp.bfloat16), (((1,), (0,)), ((), ())),
        preferred_element_type=jnp.float32)
    ranks2 = jax.lax.dot_general(
        tri, onehot2.astype(jnp.bfloat16), (((1,), (0,)), ((), ())),
        preferred_element_type=jnp.float32)

    rt1 = jnp.sum(ranks1 * onehot1, axis=1, keepdims=True).astype(jnp.int32)
    rt2 = jnp.sum(ranks2 * onehot2, axis=1, keepdims=True).astype(jnp.int32)
    kept1 = rt1 <= CAPACITY
    kept2 = rt2 <= CAPACITY
    row1 = ROW0 + (2 * top1_i) * CAPACITY + jnp.minimum(rt1 - 1, CAPACITY - 1)
    row2 = ROW0 + (2 * top2_i + 1) * CAPACITY + jnp.minimum(rt2 - 1,
                                                            CAPACITY - 1)
    d0_ref[...] = jnp.where(kept1, row1, 0)
    d1_ref[...] = jnp.where(kept2, row2, 0)
    wx0_ref[...] = jnp.broadcast_to(jnp.where(kept1, top1_v, 0.0), (N, 16))
    wx1_ref[...] = jnp.broadcast_to(jnp.where(kept2, top2_v, 0.0), (N, 16))

    c1_ref[...] = jnp.minimum(jnp.sum(onehot1, axis=0, keepdims=True),
                              float(CAPACITY)).astype(jnp.int32)
    c2_ref[...] = jnp.minimum(jnp.sum(onehot2, axis=0, keepdims=True),
                              float(CAPACITY)).astype(jnp.int32)


def _router(xf, gate_w):
    return pl.pallas_call(
        _router_body,
        out_shape=(
            jax.ShapeDtypeStruct((N, 1), jnp.int32),
            jax.ShapeDtypeStruct((N, 1), jnp.int32),
            jax.ShapeDtypeStruct((N, 16), jnp.float32),
            jax.ShapeDtypeStruct((N, 16), jnp.float32),
            jax.ShapeDtypeStruct((1, E), jnp.int32),
            jax.ShapeDtypeStruct((1, E), jnp.int32),
            jax.ShapeDtypeStruct((1, 1), jnp.float32),
            jax.ShapeDtypeStruct((1, 1), jnp.float32),
        ),
    )(xf, gate_w)


def _dispatch(xf, dest0, dest1):
    mesh = plsc.VectorSubcoreMesh(core_axis_name="c", subcore_axis_name="s")

    @functools.partial(
        pl.kernel, mesh=mesh,
        out_type=jax.ShapeDtypeStruct((XE_ROWS, C), jnp.float32),
        scratch_types=[
            pltpu.VMEM((TPW,), jnp.int32),
            pltpu.VMEM((TPW,), jnp.int32),
            pltpu.VMEM((TPW, C), jnp.float32),
            pltpu.SemaphoreType.DMA,
            pltpu.SemaphoreType.DMA,
            pltpu.SemaphoreType.DMA,
        ],
    )
    def k(xf_hbm, d0_hbm, d1_hbm, xe_hbm,
          d0_v, d1_v, xbuf, sem0, sem1, sem2):
        wid = lax.axis_index("s") * 2 + lax.axis_index("c")
        base = wid * TPW
        ld0 = pltpu.async_copy(d0_hbm.at[wid], d0_v, sem0)
        ld1 = pltpu.async_copy(d1_hbm.at[wid], d1_v, sem1)
        ldx = pltpu.async_copy(xf_hbm.at[pl.ds(base, TPW)], xbuf, sem2)
        ld0.wait()
        ld1.wait()
        ldx.wait()
        cp0 = pltpu.async_copy(xbuf, xe_hbm.at[d0_v], sem0)
        cp1 = pltpu.async_copy(xbuf, xe_hbm.at[d1_v], sem1)
        cp0.wait()
        cp1.wait()

    return k(xf, dest0, dest1)


_NB = XE_ROWS // XRB


def _expert_body(c1_ref, c2_ref, xe_ref, wg_ref, wu_ref, wd_ref, y_ref):
    b = pl.program_id(0)

    @pl.when(b == 0)
    def _():
        y_ref[...] = jnp.zeros((XRB, C), jnp.float32)

    @pl.when(b > 0)
    def _():
        seg = b - 1
        e = seg // 2
        kslot = seg % 2
        cnt = jnp.where(kslot == 0, c1_ref[e], c2_ref[e])

        @pl.when(cnt > 0)
        def _():
            xb = xe_ref[...].astype(jnp.bfloat16)
            hg = jax.lax.dot_general(
                xb, wg_ref[0].astype(jnp.bfloat16), (((1,), (1,)), ((), ())),
                preferred_element_type=jnp.float32)
            hu = jax.lax.dot_general(
                xb, wu_ref[0].astype(jnp.bfloat16), (((1,), (1,)), ((), ())),
                preferred_element_type=jnp.float32)
            silu = hg / (1.0 + jnp.exp(-hg))
            hidden = (silu * hu).astype(jnp.bfloat16)
            y_ref[...] = jax.lax.dot_general(
                hidden, wd_ref[0].astype(jnp.bfloat16),
                (((1,), (1,)), ((), ())),
                preferred_element_type=jnp.float32)


def _experts(c1, c2, xe, wg, wu, wd):
    grid_spec = pltpu.PrefetchScalarGridSpec(
        num_scalar_prefetch=2,
        grid=(_NB,),
        in_specs=[
            pl.BlockSpec((XRB, C), lambda b, c1, c2: (b, 0)),
            pl.BlockSpec((1, H, C),
                         lambda b, c1, c2: (jnp.maximum(b - 1, 0) // 2, 0, 0)),
            pl.BlockSpec((1, H, C),
                         lambda b, c1, c2: (jnp.maximum(b - 1, 0) // 2, 0, 0)),
            pl.BlockSpec((1, C, H),
                         lambda b, c1, c2: (jnp.maximum(b - 1, 0) // 2, 0, 0)),
        ],
        out_specs=pl.BlockSpec((XRB, C), lambda b, c1, c2: (b, 0)),
    )
    return pl.pallas_call(
        _expert_body,
        grid_spec=grid_spec,
        out_shape=jax.ShapeDtypeStruct((XE_ROWS, C), jnp.float32),
    )(c1, c2, xe, wg, wu, wd)


def _combine(y, row0, row1, wx0, wx1):
    mesh = plsc.VectorSubcoreMesh(core_axis_name="c", subcore_axis_name="s")

    @functools.partial(
        pl.kernel, mesh=mesh,
        out_type=jax.ShapeDtypeStruct((N, C), jnp.float32),
        scratch_types=[
            pltpu.VMEM((TPW,), jnp.int32),
            pltpu.VMEM((TPW,), jnp.int32),
            pltpu.VMEM((TPW, 16), jnp.float32),
            pltpu.VMEM((TPW, 16), jnp.float32),
            pltpu.VMEM((TPW, C), jnp.float32),
            pltpu.VMEM((TPW, C), jnp.float32),
            pltpu.SemaphoreType.DMA,
            pltpu.SemaphoreType.DMA,
        ],
    )
    def k(y_hbm, r0_hbm, r1_hbm, wx0_hbm, wx1_hbm, out_hbm,
          r0_v, r1_v, wx0_v, wx1_v, buf0, buf1, sem0, sem1):
        wid = lax.axis_index("s") * 2 + lax.axis_index("c")
        base = wid * TPW
        ld0 = pltpu.async_copy(r0_hbm.at[wid], r0_v, sem0)
        ld1 = pltpu.async_copy(r1_hbm.at[wid], r1_v, sem1)
        ldw0 = pltpu.async_copy(wx0_hbm.at[wid], wx0_v, sem0)
        ldw1 = pltpu.async_copy(wx1_hbm.at[wid], wx1_v, sem1)
        ld0.wait()
        ld1.wait()
        ldw0.wait()
        ldw1.wait()
        cp0 = pltpu.async_copy(y_hbm.at[r0_v], buf0, sem0)
        cp1 = pltpu.async_copy(y_hbm.at[r1_v], buf1, sem1)
        cp0.wait()
        cp1.wait()

        def row_body(i, carry):
            wv0 = wx0_v[i, :]
            wv1 = wx1_v[i, :]
            for j in range(C // 16):
                sl = pl.ds(j * 16, 16)
                buf0[i, sl] = wv0 * buf0[i, sl] + wv1 * buf1[i, sl]
            return carry

        lax.fori_loop(0, TPW, row_body, 0)
        pltpu.sync_copy(buf0, out_hbm.at[pl.ds(base, TPW)])

    return k(y, row0, row1, wx0, wx1)


def kernel(x, gate_w, wg, wu, wd):
    b, t, c = x.shape
    xf = x.reshape(b * t, c)
    d0, d1, wx0, wx1, c1, c2, aux, z = _router(xf, gate_w)

    xe = _dispatch(xf, d0.reshape(NW, TPW), d1.reshape(NW, TPW))

    y = _experts(c1.reshape(E), c2.reshape(E), xe, wg, wu, wd)

    out = _combine(y, d0.reshape(NW, TPW), d1.reshape(NW, TPW),
                   wx0.reshape(NW, TPW, 16), wx1.reshape(NW, TPW, 16))
    return out.reshape(b, t, c), aux[0, 0], z[0, 0]
